# baseline (speedup 1.0000x reference)
import functools

import jax
import jax.numpy as jnp
from jax import lax
from jax.experimental import pallas as pl
from jax.experimental.pallas import tpu as pltpu
from jax.experimental.pallas import tpu_sc as plsc

VOCAB = 1000000
EMB = 32
DENSE = 32
BATCH = 16384

NC = 2
NS = 16
NW = NC * NS
B_PER_W = BATCH // NW

V_BLK = 2048


def _proj_body(tablet_ref, w_ref, b_ref, out_ref):
    out_ref[...] = lax.dot_general(
        tablet_ref[...], w_ref[...],
        dimension_numbers=(((0,), (0,)), ((), ())),
        preferred_element_type=jnp.float32,
    ) + b_ref[...]


def _gather_body(idx_hbm, tw_hbm, out_hbm, idx_s, rows_v, sem):
    wid = lax.axis_index("s") * NC + lax.axis_index("c")
    base = wid * B_PER_W
    pltpu.sync_copy(idx_hbm.at[wid], idx_s)

    def issue(g, _):
        v = idx_s[pl.ds(g * 16, 16)]
        for k in range(16):
            pltpu.async_copy(tw_hbm.at[pl.ds(v[k], 1), :],
                             rows_v.at[pl.ds(g * 16 + k, 1), :], sem)
        return ()

    lax.fori_loop(0, B_PER_W // 16, issue, ())
    pltpu.make_async_copy(out_hbm.at[pl.ds(base, B_PER_W)], rows_v, sem).wait()
    pltpu.sync_copy(rows_v, out_hbm.at[pl.ds(base, B_PER_W)])


_gather = functools.partial(
    pl.kernel,
    mesh=plsc.VectorSubcoreMesh(core_axis_name="c", subcore_axis_name="s"),
    out_type=jax.ShapeDtypeStruct((BATCH, DENSE), jnp.float32),
    scratch_types=[
        pltpu.VMEM((B_PER_W,), jnp.int32),
        pltpu.VMEM((B_PER_W, DENSE), jnp.float32),
        pltpu.SemaphoreType.DMA,
    ],
)(_gather_body)


def kernel(indices, table, W, b):
    idx2 = indices.astype(jnp.int32).reshape(NW, B_PER_W)
    tw = pl.pallas_call(
        _proj_body,
        grid=(pl.cdiv(VOCAB, V_BLK),),
        in_specs=[
            pl.BlockSpec((EMB, V_BLK), lambda i: (0, i)),
            pl.BlockSpec((EMB, DENSE), lambda i: (0, 0)),
            pl.BlockSpec((1, DENSE), lambda i: (0, 0)),
        ],
        out_specs=pl.BlockSpec((V_BLK, DENSE), lambda i: (i, 0)),
        out_shape=jax.ShapeDtypeStruct((VOCAB, DENSE), jnp.float32),
    )(table.T, W, b.reshape(1, DENSE))
    return _gather(idx2, tw)


# speedup vs baseline: 1.0888x; 1.0888x over previous
import functools

import jax
import jax.numpy as jnp
from jax import lax
from jax.experimental import pallas as pl
from jax.experimental.pallas import tpu as pltpu
from jax.experimental.pallas import tpu_sc as plsc

VOCAB = 1000000
EMB = 32
DENSE = 32
BATCH = 16384

NC = 2
NS = 16
NW = NC * NS
B_PER_W = BATCH // NW

PACK = 128 // DENSE
V_BLK = 2048


def _proj_body(tablet_ref, w_ref, b_ref, out_ref):
    res = lax.dot_general(
        tablet_ref[...], w_ref[...],
        dimension_numbers=(((0,), (0,)), ((), ())),
        preferred_element_type=jnp.float32,
    ) + b_ref[...]
    q = V_BLK // PACK
    out_ref[...] = jnp.concatenate(
        [res[j * q:(j + 1) * q, :] for j in range(PACK)], axis=1)


HALF = B_PER_W // 2


def _gather_body(idx_hbm, tw_hbm, out_hbm, idx_s, rows4_v, out_v, sem):
    wid = lax.axis_index("s") * NC + lax.axis_index("c")
    base = wid * B_PER_W
    pltpu.sync_copy(idx_hbm.at[wid], idx_s)

    for h in range(2):
        def issue(g, _, h=h):
            v = idx_s[pl.ds(h * HALF + g * 16, 16)]
            for k in range(16):
                r = v[k]
                p = (lax.shift_right_logical(r, 11) * (V_BLK // PACK)
                     + (r & (V_BLK // PACK - 1)))
                pltpu.async_copy(tw_hbm.at[pl.ds(p, 1), :],
                                 rows4_v.at[pl.ds(g * 16 + k, 1), :], sem)
            return ()

        lax.fori_loop(0, HALF // 16, issue, ())
        pltpu.make_async_copy(tw_hbm.at[pl.ds(0, HALF), :], rows4_v, sem).wait()

        def extract(g, _, h=h):
            v = idx_s[pl.ds(h * HALF + g * 16, 16)]
            for k in range(16):
                o = (lax.shift_right_logical(v[k], 9) & (PACK - 1)) * DENSE
                i = g * 16 + k
                j = h * HALF + i
                out_v[j, pl.ds(0, 16)] = rows4_v[i, pl.ds(o, 16)]
                out_v[j, pl.ds(16, 16)] = rows4_v[i, pl.ds(o + 16, 16)]
            return ()

        lax.fori_loop(0, HALF // 16, extract, ())

    pltpu.sync_copy(out_v, out_hbm.at[pl.ds(base, B_PER_W)])


_gather = functools.partial(
    pl.kernel,
    mesh=plsc.VectorSubcoreMesh(core_axis_name="c", subcore_axis_name="s"),
    out_type=jax.ShapeDtypeStruct((BATCH, DENSE), jnp.float32),
    scratch_types=[
        pltpu.VMEM((B_PER_W,), jnp.int32),
        pltpu.VMEM((HALF, DENSE * PACK), jnp.float32),
        pltpu.VMEM((B_PER_W, DENSE), jnp.float32),
        pltpu.SemaphoreType.DMA,
    ],
)(_gather_body)


def kernel(indices, table, W, b):
    idx2 = indices.astype(jnp.int32).reshape(NW, B_PER_W)
    tw4 = pl.pallas_call(
        _proj_body,
        grid=(pl.cdiv(VOCAB, V_BLK),),
        in_specs=[
            pl.BlockSpec((EMB, V_BLK), lambda i: (0, i)),
            pl.BlockSpec((EMB, DENSE), lambda i: (0, 0)),
            pl.BlockSpec((1, DENSE), lambda i: (0, 0)),
        ],
        out_specs=pl.BlockSpec((V_BLK // PACK, DENSE * PACK), lambda i: (i, 0)),
        out_shape=jax.ShapeDtypeStruct(
            (pl.cdiv(VOCAB, V_BLK) * (V_BLK // PACK), DENSE * PACK),
            jnp.float32),
    )(table.T, W, b.reshape(1, DENSE))
    return _gather(idx2, tw4)


# speedup vs baseline: 1.7030x; 1.5641x over previous
import functools

import jax
import jax.numpy as jnp
from jax import lax
from jax.experimental import pallas as pl
from jax.experimental.pallas import tpu as pltpu
from jax.experimental.pallas import tpu_sc as plsc

VOCAB = 1000000
EMB = 32
DENSE = 32
BATCH = 16384

NC = 2
NS = 16
NW = NC * NS
B_PER_W = BATCH // NW

PACK = 128 // DENSE
V_SUB = 2048
N_SUB = 4
V_BLK = V_SUB * N_SUB
Q = V_SUB // PACK


def _proj_body(tablet_ref, w_ref, b_ref, out_ref):
    for u in range(N_SUB):
        res = lax.dot_general(
            tablet_ref[:, u * V_SUB:(u + 1) * V_SUB], w_ref[...],
            dimension_numbers=(((0,), (0,)), ((), ())),
            preferred_element_type=jnp.float32,
        ) + b_ref[...]
        out_ref[u * Q:(u + 1) * Q, :] = jnp.concatenate(
            [res[j * Q:(j + 1) * Q, :] for j in range(PACK)], axis=1)


HALF = B_PER_W // 2


def _gather_body(idx_hbm, tw_hbm, out_hbm, idx_s, rows4_v, out_v, sem):
    wid = lax.axis_index("s") * NC + lax.axis_index("c")
    base = wid * B_PER_W
    pltpu.sync_copy(idx_hbm.at[wid], idx_s)

    for h in range(2):
        def issue(g, _, h=h):
            v = idx_s[pl.ds(h * HALF + g * 16, 16)]
            for k in range(16):
                r = v[k]
                p = lax.shift_right_logical(r, 11) * Q + (r & (Q - 1))
                pltpu.async_copy(tw_hbm.at[pl.ds(p, 1), :],
                                 rows4_v.at[pl.ds(g * 16 + k, 1), :], sem)
            return ()

        lax.fori_loop(0, HALF // 16, issue, ())
        pltpu.make_async_copy(tw_hbm.at[pl.ds(0, HALF), :], rows4_v, sem).wait()

        def extract(g, _, h=h):
            v = idx_s[pl.ds(h * HALF + g * 16, 16)]
            for k in range(16):
                o = (lax.shift_right_logical(v[k], 9) & (PACK - 1)) * DENSE
                i = g * 16 + k
                j = h * HALF + i
                out_v[j, pl.ds(0, 16)] = rows4_v[i, pl.ds(o, 16)]
                out_v[j, pl.ds(16, 16)] = rows4_v[i, pl.ds(o + 16, 16)]
            return ()

        lax.fori_loop(0, HALF // 16, extract, ())

    pltpu.sync_copy(out_v, out_hbm.at[pl.ds(base, B_PER_W)])


_gather = functools.partial(
    pl.kernel,
    mesh=plsc.VectorSubcoreMesh(core_axis_name="c", subcore_axis_name="s"),
    out_type=jax.ShapeDtypeStruct((BATCH, DENSE), jnp.float32),
    scratch_types=[
        pltpu.VMEM((B_PER_W,), jnp.int32),
        pltpu.VMEM((HALF, DENSE * PACK), jnp.float32),
        pltpu.VMEM((B_PER_W, DENSE), jnp.float32),
        pltpu.SemaphoreType.DMA,
    ],
)(_gather_body)


def kernel(indices, table, W, b):
    idx2 = indices.astype(jnp.int32).reshape(NW, B_PER_W)
    tw4 = pl.pallas_call(
        _proj_body,
        grid=(pl.cdiv(VOCAB, V_BLK),),
        in_specs=[
            pl.BlockSpec((EMB, V_BLK), lambda i: (0, i)),
            pl.BlockSpec((EMB, DENSE), lambda i: (0, 0)),
            pl.BlockSpec((1, DENSE), lambda i: (0, 0)),
        ],
        out_specs=pl.BlockSpec((V_BLK // PACK, DENSE * PACK), lambda i: (i, 0)),
        out_shape=jax.ShapeDtypeStruct(
            (pl.cdiv(VOCAB, V_BLK) * (V_BLK // PACK), DENSE * PACK),
            jnp.float32),
    )(table.T, W, b.reshape(1, DENSE))
    return _gather(idx2, tw4)


# speedup vs baseline: 2.0524x; 1.2052x over previous
import functools

import jax
import jax.numpy as jnp
from jax import lax
from jax.experimental import pallas as pl
from jax.experimental.pallas import tpu as pltpu
from jax.experimental.pallas import tpu_sc as plsc

VOCAB = 1000000
EMB = 32
DENSE = 32
BATCH = 16384

NC = 2
NS = 16
NW = NC * NS
B_PER_W = BATCH // NW

PACK = 128 // DENSE
V_SUB = 2048
N_SUB = 4
V_BLK = V_SUB * N_SUB
Q = V_SUB // PACK


def _proj_body(tablet_ref, w_ref, b_ref, out_ref):
    wb = w_ref[...].astype(jnp.bfloat16)
    for u in range(N_SUB):
        res = lax.dot_general(
            tablet_ref[:, u * V_SUB:(u + 1) * V_SUB].astype(jnp.bfloat16),
            wb,
            dimension_numbers=(((0,), (0,)), ((), ())),
            preferred_element_type=jnp.float32,
        ) + b_ref[...]
        out_ref[u * Q:(u + 1) * Q, :] = jnp.concatenate(
            [res[j * Q:(j + 1) * Q, :] for j in range(PACK)], axis=1)


HALF = B_PER_W // 2


def _gather_body(idx_hbm, tw_hbm, out_hbm, idx_s, rows4_v, out_v, sem):
    wid = lax.axis_index("s") * NC + lax.axis_index("c")
    base = wid * B_PER_W
    pltpu.sync_copy(idx_hbm.at[wid], idx_s)

    for h in range(2):
        def issue(g, _, h=h):
            v = idx_s[pl.ds(h * HALF + g * 16, 16)]
            for k in range(16):
                r = v[k]
                p = lax.shift_right_logical(r, 11) * Q + (r & (Q - 1))
                pltpu.async_copy(tw_hbm.at[pl.ds(p, 1), :],
                                 rows4_v.at[pl.ds(g * 16 + k, 1), :], sem)
            return ()

        lax.fori_loop(0, HALF // 16, issue, ())
        pltpu.make_async_copy(tw_hbm.at[pl.ds(0, HALF), :], rows4_v, sem).wait()

        def extract(g, _, h=h):
            v = idx_s[pl.ds(h * HALF + g * 16, 16)]
            for k in range(16):
                o = (lax.shift_right_logical(v[k], 9) & (PACK - 1)) * DENSE
                i = g * 16 + k
                j = h * HALF + i
                out_v[j, pl.ds(0, 16)] = rows4_v[i, pl.ds(o, 16)]
                out_v[j, pl.ds(16, 16)] = rows4_v[i, pl.ds(o + 16, 16)]
            return ()

        lax.fori_loop(0, HALF // 16, extract, ())

    pltpu.sync_copy(out_v, out_hbm.at[pl.ds(base, B_PER_W)])


_gather = functools.partial(
    pl.kernel,
    mesh=plsc.VectorSubcoreMesh(core_axis_name="c", subcore_axis_name="s"),
    out_type=jax.ShapeDtypeStruct((BATCH, DENSE), jnp.float32),
    scratch_types=[
        pltpu.VMEM((B_PER_W,), jnp.int32),
        pltpu.VMEM((HALF, DENSE * PACK), jnp.float32),
        pltpu.VMEM((B_PER_W, DENSE), jnp.float32),
        pltpu.SemaphoreType.DMA,
    ],
)(_gather_body)


def kernel(indices, table, W, b):
    idx2 = indices.astype(jnp.int32).reshape(NW, B_PER_W)
    tw4 = pl.pallas_call(
        _proj_body,
        grid=(pl.cdiv(VOCAB, V_BLK),),
        in_specs=[
            pl.BlockSpec((EMB, V_BLK), lambda i: (0, i)),
            pl.BlockSpec((EMB, DENSE), lambda i: (0, 0)),
            pl.BlockSpec((1, DENSE), lambda i: (0, 0)),
        ],
        out_specs=pl.BlockSpec((V_BLK // PACK, DENSE * PACK), lambda i: (i, 0)),
        out_shape=jax.ShapeDtypeStruct(
            (pl.cdiv(VOCAB, V_BLK) * (V_BLK // PACK), DENSE * PACK),
            jnp.float32),
    )(table.T, W, b.reshape(1, DENSE))
    return _gather(idx2, tw4)


# speedup vs baseline: 2.2341x; 1.0885x over previous
import functools

import jax
import jax.numpy as jnp
from jax import lax
from jax.experimental import pallas as pl
from jax.experimental.pallas import tpu as pltpu
from jax.experimental.pallas import tpu_sc as plsc

VOCAB = 1000000
EMB = 32
DENSE = 32
BATCH = 16384

NC = 2
NS = 16
NW = NC * NS
B_PER_W = BATCH // NW

PACK = 128 // DENSE
V_SUB = 1024
N_SUB = 16
V_BLK = V_SUB * N_SUB
Q = V_SUB // PACK
SH_P = V_SUB.bit_length() - 1
SH_O = Q.bit_length() - 1


def _proj_body(tablet_ref, w_ref, b_ref, out_ref):
    wb = w_ref[...].astype(jnp.bfloat16)
    for u in range(N_SUB):
        res = lax.dot_general(
            tablet_ref[:, u * V_SUB:(u + 1) * V_SUB].astype(jnp.bfloat16),
            wb,
            dimension_numbers=(((0,), (0,)), ((), ())),
            preferred_element_type=jnp.float32,
        ) + b_ref[...]
        out_ref[u * Q:(u + 1) * Q, :] = jnp.concatenate(
            [res[j * Q:(j + 1) * Q, :] for j in range(PACK)], axis=1)


HALF = B_PER_W // 2


def _gather_body(idx_hbm, tw_hbm, out_hbm, idx_s, rows4_v, out_v, sem):
    wid = lax.axis_index("s") * NC + lax.axis_index("c")
    base = wid * B_PER_W
    pltpu.sync_copy(idx_hbm.at[wid], idx_s)

    for h in range(2):
        def issue(g, _, h=h):
            v = idx_s[pl.ds(h * HALF + g * 16, 16)]
            for k in range(16):
                r = v[k]
                p = lax.shift_right_logical(r, SH_P) * Q + (r & (Q - 1))
                pltpu.async_copy(tw_hbm.at[pl.ds(p, 1), :],
                                 rows4_v.at[pl.ds(g * 16 + k, 1), :], sem)
            return ()

        lax.fori_loop(0, HALF // 16, issue, ())
        pltpu.make_async_copy(tw_hbm.at[pl.ds(0, HALF), :], rows4_v, sem).wait()

        def extract(g, _, h=h):
            v = idx_s[pl.ds(h * HALF + g * 16, 16)]
            for k in range(16):
                o = (lax.shift_right_logical(v[k], SH_O) & (PACK - 1)) * DENSE
                i = g * 16 + k
                j = h * HALF + i
                out_v[j, pl.ds(0, 16)] = rows4_v[i, pl.ds(o, 16)]
                out_v[j, pl.ds(16, 16)] = rows4_v[i, pl.ds(o + 16, 16)]
            return ()

        lax.fori_loop(0, HALF // 16, extract, ())

    pltpu.sync_copy(out_v, out_hbm.at[pl.ds(base, B_PER_W)])


_gather = functools.partial(
    pl.kernel,
    mesh=plsc.VectorSubcoreMesh(core_axis_name="c", subcore_axis_name="s"),
    out_type=jax.ShapeDtypeStruct((BATCH, DENSE), jnp.float32),
    scratch_types=[
        pltpu.VMEM((B_PER_W,), jnp.int32),
        pltpu.VMEM((HALF, DENSE * PACK), jnp.float32),
        pltpu.VMEM((B_PER_W, DENSE), jnp.float32),
        pltpu.SemaphoreType.DMA,
    ],
)(_gather_body)


def kernel(indices, table, W, b):
    idx2 = indices.astype(jnp.int32).reshape(NW, B_PER_W)
    tw4 = pl.pallas_call(
        _proj_body,
        grid=(pl.cdiv(VOCAB, V_BLK),),
        in_specs=[
            pl.BlockSpec((EMB, V_BLK), lambda i: (0, i)),
            pl.BlockSpec((EMB, DENSE), lambda i: (0, 0)),
            pl.BlockSpec((1, DENSE), lambda i: (0, 0)),
        ],
        out_specs=pl.BlockSpec((V_BLK // PACK, DENSE * PACK), lambda i: (i, 0)),
        out_shape=jax.ShapeDtypeStruct(
            (pl.cdiv(VOCAB, V_BLK) * (V_BLK // PACK), DENSE * PACK),
            jnp.float32),
    )(table.T, W, b.reshape(1, DENSE))
    return _gather(idx2, tw4)
